# baseline (speedup 1.0000x reference)
import jax
import jax.numpy as jnp
from jax import lax
from jax.experimental import pallas as pl
from jax.experimental.pallas import tpu as pltpu
from jax.experimental.pallas import tpu_sc as plsc

N_NODES = 50000
N_EDGES = 800000
T = 8
F = 64
U = 64
H = 64
NUM_CLASSES = 40
BATCH = 1024
BN_SCALE = float(1.0 / (1.0 + 1e-3) ** 0.5)

NC = 2
NS = 16
HALF = H // 2
CHUNK = 128
TILE_CHUNKS = 400
GROUP_PAIRS = TILE_CHUNKS // 16
TILE_ROWS = TILE_CHUNKS
EDGES_PAD = NS * TILE_CHUNKS * CHUNK
NODES_PAD = 50048
STRIPE = NODES_PAD // NS
BK = 1000


def _gelu(x):
    return 0.5 * x * (1.0 + lax.erf(x * (2.0 ** -0.5)))


def _lstm_prep_body(nf_ref, wk_ref, wr_ref, b_ref, wp_ref, bp_ref, ew_ref,
                    x1_ref, ma_ref, mb_ref, s_ref):
    @pl.when(pl.program_id(0) == 0)
    def _():
        s_ref[...] = jnp.sum(ew_ref[...]).reshape(1, 1)

    wk = wk_ref[...]
    wr = wr_ref[...]
    b = b_ref[...]
    h = jnp.zeros((BK, U), jnp.float32)
    c = jnp.zeros((BK, U), jnp.float32)
    for t in range(T):
        xt = nf_ref[:, t * F:(t + 1) * F]
        z = (jnp.dot(xt, wk, preferred_element_type=jnp.float32)
             + jnp.dot(h, wr, preferred_element_type=jnp.float32) + b)
        i = jax.nn.sigmoid(z[:, 0:U])
        f = jax.nn.sigmoid(z[:, U:2 * U])
        g = jnp.tanh(z[:, 2 * U:3 * U])
        o = jax.nn.sigmoid(z[:, 3 * U:4 * U])
        c = f * c + i * g
        h = o * jnp.tanh(c)
    x1_ref[...] = h
    m = _gelu(jnp.dot(h * BN_SCALE, wp_ref[...],
                      preferred_element_type=jnp.float32) + bp_ref[...])
    ma_ref[...] = m[:, :HALF]
    mb_ref[...] = m[:, HALF:]


def _lstm_prep(nf, wk, wr, b, wp, bp, ew2d):
    grid = (N_NODES // BK,)
    return pl.pallas_call(
        _lstm_prep_body,
        grid=grid,
        in_specs=[
            pl.BlockSpec((BK, T * F), lambda i: (i, 0)),
            pl.BlockSpec((F, 4 * U), lambda i: (0, 0)),
            pl.BlockSpec((U, 4 * U), lambda i: (0, 0)),
            pl.BlockSpec((1, 4 * U), lambda i: (0, 0)),
            pl.BlockSpec((U, H), lambda i: (0, 0)),
            pl.BlockSpec((1, H), lambda i: (0, 0)),
            pl.BlockSpec((3125, 256), lambda i: (0, 0)),
        ],
        out_specs=[
            pl.BlockSpec((BK, U), lambda i: (i, 0)),
            pl.BlockSpec((BK, HALF), lambda i: (i, 0)),
            pl.BlockSpec((BK, HALF), lambda i: (i, 0)),
            pl.BlockSpec((1, 1), lambda i: (0, 0)),
        ],
        out_shape=[
            jax.ShapeDtypeStruct((N_NODES, U), jnp.float32),
            jax.ShapeDtypeStruct((N_NODES, HALF), jnp.float32),
            jax.ShapeDtypeStruct((N_NODES, HALF), jnp.float32),
            jax.ShapeDtypeStruct((1, 1), jnp.float32),
        ],
    )(nf, wk, wr, b, wp, bp, ew2d)


def _sc_conv_body(ma, mb, src2d, dst2d, w1d, outa, outb,
                  rows0, rows1, rows2, rows3,
                  gsrcA, gdstA, gwA, gsrcB, gdstB, gwB,
                  aggsm,
                  semG0, semG1, semG2, semG3,
                  semS0, semS1, semS2, semS3, semLA, semLB):
    c = lax.axis_index("c")
    s = lax.axis_index("s")
    base = s * STRIPE
    row0 = s * TILE_ROWS
    rows = [rows0, rows1, rows2, rows3]
    semG = [semG0, semG1, semG2, semG3]
    semS = [semS0, semS1, semS2, semS3]

    def conv(m_hbm, out_hbm):
        @pl.loop(0, 128)
        def _(i):
            z16 = jnp.zeros((16,), jnp.float32)
            rows0[i, pl.ds(0, 16)] = z16
            rows0[i, pl.ds(16, 16)] = z16

        for q in range(24):
            pltpu.async_copy(rows0, aggsm.at[pl.ds(base + q * 128, 128), :],
                             semG0)
        pltpu.async_copy(rows0.at[pl.ds(0, 56), :],
                         aggsm.at[pl.ds(base + 3072, 56), :], semG0)
        for q in range(24):
            pltpu.make_async_copy(
                rows0, aggsm.at[pl.ds(base + q * 128, 128), :], semG0).wait()
        pltpu.make_async_copy(
            rows0.at[pl.ds(0, 56), :],
            aggsm.at[pl.ds(base + 3072, 56), :], semG0).wait()
        plsc.subcore_barrier()

        def fire_lin(g, gsrc, gdst, gw, semL):
            r = row0 + g * 8
            pltpu.async_copy(src2d.at[pl.ds(r, 8), :], gsrc, semL)
            pltpu.async_copy(dst2d.at[pl.ds(r, 8), :], gdst, semL)
            pltpu.async_copy(w1d.at[pl.ds(r * 128, 1024)], gw, semL)

        def wait_lin(g, gsrc, gdst, gw, semL):
            r = row0 + g * 8
            pltpu.make_async_copy(src2d.at[pl.ds(r, 8), :], gsrc, semL).wait()
            pltpu.make_async_copy(dst2d.at[pl.ds(r, 8), :], gdst, semL).wait()
            pltpu.make_async_copy(w1d.at[pl.ds(r * 128, 1024)], gw, semL).wait()

        def fire_gather(gsrc, ir, r):
            pltpu.async_copy(m_hbm.at[gsrc.at[ir]], rows[r], semG[r])

        def wait_gather(gsrc, ir, r):
            pltpu.make_async_copy(m_hbm.at[gsrc.at[ir]], rows[r],
                                  semG[r]).wait()

        def fire_scatter(gdst, ir, r):
            pltpu.async_copy(rows[r], aggsm.at[gdst.at[ir]], semS[r],
                             add=True)

        def wait_scatter(gdst, ir, r):
            pltpu.make_async_copy(rows[r], aggsm.at[gdst.at[ir]],
                                  semS[r]).wait()

        def scale(gw, off, r):
            rbuf = rows[r]

            @pl.loop(0, 16)
            def _(blk):
                b8 = blk * 8
                for t in range(8):
                    e = b8 + t
                    wv = plsc.load_gather(
                        gw, [jnp.full((16,), off + e, jnp.int32)])
                    lo = rbuf[e, pl.ds(0, 16)]
                    hi = rbuf[e, pl.ds(16, 16)]
                    rbuf[e, pl.ds(0, 16)] = lo * wv
                    rbuf[e, pl.ds(16, 16)] = hi * wv

        fire_lin(0, gsrcA, gdstA, gwA, semLA)
        wait_lin(0, gsrcA, gdstA, gwA, semLA)
        for i in range(3):
            fire_gather(gsrcA, i, i)

        @pl.loop(0, GROUP_PAIRS)
        def _(k):
            for i in range(16):
                if i < 8:
                    gsrc_i, gdst_i, gw_i = gsrcA, gdstA, gwA
                else:
                    gsrc_i, gdst_i, gw_i = gsrcB, gdstB, gwB
                r = i & 3
                ir = i & 7
                wait_gather(gsrc_i, ir, r)
                scale(gw_i, ir * 128, r)
                fire_scatter(gdst_i, ir, r)
                i3 = i + 3
                r3 = i3 & 3
                ir3 = i3 & 7
                if i3 < 8:
                    g3src = gsrcA
                elif i3 < 16:
                    g3src = gsrcB
                else:
                    g3src = gsrcA
                if i == 0:
                    pdst = gdstB
                    pir = 7
                elif i == 8:
                    pdst = gdstA
                    pir = 7
                else:
                    pdst = gdst_i
                    pir = ir - 1
                if i == 0:
                    @pl.when(k > 0)
                    def _():
                        wait_scatter(pdst, pir, r3)
                    fire_gather(g3src, ir3, r3)
                    fire_lin(2 * k + 1, gsrcB, gdstB, gwB, semLB)
                elif i < 13:
                    wait_scatter(pdst, pir, r3)
                    if i == 5:
                        wait_lin(2 * k + 1, gsrcB, gdstB, gwB, semLB)
                    fire_gather(g3src, ir3, r3)
                    if i == 8:
                        @pl.when(k < GROUP_PAIRS - 1)
                        def _():
                            fire_lin(2 * k + 2, gsrcA, gdstA, gwA, semLA)
                else:
                    @pl.when(k < GROUP_PAIRS - 1)
                    def _():
                        wait_scatter(pdst, pir, r3)
                        if i == 13:
                            wait_lin(2 * k + 2, gsrcA, gdstA, gwA, semLA)
                        fire_gather(g3src, ir3, r3)

        for ir in range(4, 8):
            wait_scatter(gdstB, ir, ir & 3)

        plsc.subcore_barrier()

        for q in range(24):
            r = q & 3
            if q >= 4:
                pltpu.make_async_copy(
                    rows[r], out_hbm.at[pl.ds(base + (q - 4) * 128, 128), :],
                    semS[r]).wait()
            pltpu.sync_copy(aggsm.at[pl.ds(base + q * 128, 128), :], rows[r])
            pltpu.async_copy(rows[r],
                             out_hbm.at[pl.ds(base + q * 128, 128), :],
                             semS[r])
        pltpu.make_async_copy(
            rows[0], out_hbm.at[pl.ds(base + 20 * 128, 128), :], semS[0]).wait()
        pltpu.sync_copy(aggsm.at[pl.ds(base + 3072, 56), :],
                        rows0.at[pl.ds(0, 56), :])
        pltpu.async_copy(rows0.at[pl.ds(0, 56), :],
                         out_hbm.at[pl.ds(base + 3072, 56), :], semS[0])
        for q in (21, 22, 23):
            pltpu.make_async_copy(
                rows[q & 3], out_hbm.at[pl.ds(base + q * 128, 128), :],
                semS[q & 3]).wait()
        pltpu.make_async_copy(
            rows0.at[pl.ds(0, 56), :],
            out_hbm.at[pl.ds(base + 3072, 56), :], semS[0]).wait()

    @pl.when(c == 0)
    def _():
        conv(ma, outa)

    @pl.when(c == 1)
    def _():
        conv(mb, outb)


def _sc_conv(ma, mb, src2d, dst2d, w1d):
    mesh = plsc.VectorSubcoreMesh(core_axis_name="c", subcore_axis_name="s")
    f = pl.kernel(
        _sc_conv_body,
        out_type=(jax.ShapeDtypeStruct((NODES_PAD, HALF), jnp.float32),
                  jax.ShapeDtypeStruct((NODES_PAD, HALF), jnp.float32)),
        mesh=mesh,
        scratch_types=(
            [pltpu.VMEM((128, HALF), jnp.float32) for _ in range(4)]
            + [pltpu.VMEM((8, 128), jnp.int32), pltpu.VMEM((8, 128), jnp.int32),
               pltpu.VMEM((1024,), jnp.float32)]
            + [pltpu.VMEM((8, 128), jnp.int32), pltpu.VMEM((8, 128), jnp.int32),
               pltpu.VMEM((1024,), jnp.float32)]
            + [pltpu.VMEM_SHARED((NODES_PAD, HALF), jnp.float32)]
            + [pltpu.SemaphoreType.DMA] * 10
        ),
        compiler_params=pltpu.CompilerParams(needs_layout_passes=False, use_tc_tiling_on_sc=False),
    )
    return f(ma, mb, src2d, dst2d, w1d)


def _update_body(x_ref, aa_ref, ab_ref, s_ref, wu_ref, bu_ref, wp_ref, bp_ref,
                 x2_ref, ma_ref, mb_ref):
    inv = BN_SCALE / s_ref[...]
    x = x_ref[...]
    wu = wu_ref[...]
    z = (BN_SCALE * jnp.dot(x, wu[:U, :], preferred_element_type=jnp.float32)
         + inv * jnp.dot(aa_ref[...], wu[U:U + HALF, :],
                         preferred_element_type=jnp.float32)
         + inv * jnp.dot(ab_ref[...], wu[U + HALF:, :],
                         preferred_element_type=jnp.float32)
         + bu_ref[...])
    out = _gelu(z)
    sq = jnp.maximum(jnp.sum(out * out, axis=-1, keepdims=True), 1e-12)
    x2 = out * lax.rsqrt(sq)
    x2_ref[...] = x2
    m = _gelu(jnp.dot(x2 * BN_SCALE, wp_ref[...],
                      preferred_element_type=jnp.float32) + bp_ref[...])
    ma_ref[...] = m[:, :HALF]
    mb_ref[...] = m[:, HALF:]


def _update(x1, aa, ab, s, wu, bu, wp, bp):
    grid = (N_NODES // BK,)
    return pl.pallas_call(
        _update_body,
        grid=grid,
        in_specs=[
            pl.BlockSpec((BK, U), lambda i: (i, 0)),
            pl.BlockSpec((BK, HALF), lambda i: (i, 0)),
            pl.BlockSpec((BK, HALF), lambda i: (i, 0)),
            pl.BlockSpec((1, 1), lambda i: (0, 0)),
            pl.BlockSpec((U + H, H), lambda i: (0, 0)),
            pl.BlockSpec((1, H), lambda i: (0, 0)),
            pl.BlockSpec((H, H), lambda i: (0, 0)),
            pl.BlockSpec((1, H), lambda i: (0, 0)),
        ],
        out_specs=[
            pl.BlockSpec((BK, H), lambda i: (i, 0)),
            pl.BlockSpec((BK, HALF), lambda i: (i, 0)),
            pl.BlockSpec((BK, HALF), lambda i: (i, 0)),
        ],
        out_shape=[
            jax.ShapeDtypeStruct((N_NODES, H), jnp.float32),
            jax.ShapeDtypeStruct((N_NODES, HALF), jnp.float32),
            jax.ShapeDtypeStruct((N_NODES, HALF), jnp.float32),
        ],
    )(x1, aa, ab, s, wu, bu, wp, bp)


def _sc_gather_body(x2, a2, b2, idx, ox, oa, ob, idxb, rx, ra, rb, sem):
    c = lax.axis_index("c")
    s = lax.axis_index("s")
    wid = s * NC + c
    per = BATCH // (NC * NS)
    base = wid * per
    pltpu.sync_copy(idx.at[pl.ds(base, per)], idxb)
    d1 = pltpu.async_copy(x2.at[idxb], rx, sem)
    d2 = pltpu.async_copy(a2.at[idxb], ra, sem)
    d3 = pltpu.async_copy(b2.at[idxb], rb, sem)
    d1.wait()
    d2.wait()
    d3.wait()
    pltpu.sync_copy(rx, ox.at[pl.ds(base, per), :])
    pltpu.sync_copy(ra, oa.at[pl.ds(base, per), :])
    pltpu.sync_copy(rb, ob.at[pl.ds(base, per), :])


def _sc_gather(x2, a2, b2, idx):
    per = BATCH // (NC * NS)
    mesh = plsc.VectorSubcoreMesh(core_axis_name="c", subcore_axis_name="s")
    f = pl.kernel(
        _sc_gather_body,
        out_type=(jax.ShapeDtypeStruct((BATCH, H), jnp.float32),
                  jax.ShapeDtypeStruct((BATCH, HALF), jnp.float32),
                  jax.ShapeDtypeStruct((BATCH, HALF), jnp.float32)),
        mesh=mesh,
        scratch_types=[
            pltpu.VMEM((per,), jnp.int32),
            pltpu.VMEM((per, H), jnp.float32),
            pltpu.VMEM((per, HALF), jnp.float32),
            pltpu.VMEM((per, HALF), jnp.float32),
            pltpu.SemaphoreType.DMA,
        ],
        compiler_params=pltpu.CompilerParams(needs_layout_passes=False, use_tc_tiling_on_sc=False),
    )
    return f(x2, a2, b2, idx)


def _final_body(x_ref, aa_ref, ab_ref, s_ref, wu_ref, bu_ref,
                wpo_ref, bpo_ref, wl_ref, bl_ref, out_ref):
    inv = BN_SCALE / s_ref[...]
    x = x_ref[...]
    wu = wu_ref[...]
    z = (BN_SCALE * jnp.dot(x, wu[:H, :], preferred_element_type=jnp.float32)
         + inv * jnp.dot(aa_ref[...], wu[H:H + HALF, :],
                         preferred_element_type=jnp.float32)
         + inv * jnp.dot(ab_ref[...], wu[H + HALF:, :],
                         preferred_element_type=jnp.float32)
         + bu_ref[...])
    out = _gelu(z)
    sq = jnp.maximum(jnp.sum(out * out, axis=-1, keepdims=True), 1e-12)
    x3 = out * lax.rsqrt(sq)
    x4 = _gelu(jnp.dot(x3 * BN_SCALE, wpo_ref[...],
                       preferred_element_type=jnp.float32) + bpo_ref[...])
    out_ref[...] = (jnp.dot(x4, wl_ref[...], preferred_element_type=jnp.float32)
                    + bl_ref[...])


def _final(xg, ag, bg, s, wu, bu, wpo, bpo, wl, bl):
    return pl.pallas_call(
        _final_body,
        out_shape=jax.ShapeDtypeStruct((BATCH, NUM_CLASSES), jnp.float32),
    )(xg, ag, bg, s, wu, bu, wpo, bpo, wl, bl)


def kernel(node_features, edge_weights, lstm_kernel, lstm_recurrent, lstm_bias,
           c1_prep_W, c1_prep_b, c1_upd_W, c1_upd_b,
           c2_prep_W, c2_prep_b, c2_upd_W, c2_upd_b,
           post_W, post_b, logits_W, logits_b, edges, input_node_indices):
    nf = node_features.reshape(N_NODES, T * F)
    dst = edges[0]
    src = edges[1]
    pad = EDGES_PAD - N_EDGES
    src2d = jnp.concatenate([src, jnp.zeros((pad,), jnp.int32)]).reshape(-1, 128)
    dst2d = jnp.concatenate([dst, jnp.zeros((pad,), jnp.int32)]).reshape(-1, 128)
    w1d = jnp.concatenate([edge_weights, jnp.zeros((pad,), jnp.float32)])

    wu1 = c1_upd_W
    wu2 = c2_upd_W

    x1, m1a, m1b, s = _lstm_prep(nf, lstm_kernel, lstm_recurrent,
                                 lstm_bias.reshape(1, -1),
                                 c1_prep_W, c1_prep_b.reshape(1, -1),
                                 edge_weights.reshape(3125, 256))
    a1a, a1b = _sc_conv(m1a, m1b, src2d, dst2d, w1d)
    x2, m2a, m2b = _update(x1, a1a, a1b, s, wu1,
                           c1_upd_b.reshape(1, -1),
                           c2_prep_W, c2_prep_b.reshape(1, -1))
    a2a, a2b = _sc_conv(m2a, m2b, src2d, dst2d, w1d)
    xg, ag, bg = _sc_gather(x2, a2a, a2b, input_node_indices)
    return _final(xg, ag, bg, s, wu2, c2_upd_b.reshape(1, -1),
                  post_W, post_b.reshape(1, -1),
                  logits_W, logits_b.reshape(1, -1))


# speedup vs baseline: 1.1375x; 1.1375x over previous
import jax
import jax.numpy as jnp
from jax import lax
from jax.experimental import pallas as pl
from jax.experimental.pallas import tpu as pltpu
from jax.experimental.pallas import tpu_sc as plsc

N_NODES = 50000
N_EDGES = 800000
T = 8
F = 64
U = 64
H = 64
NUM_CLASSES = 40
BATCH = 1024
BN_SCALE = float(1.0 / (1.0 + 1e-3) ** 0.5)

NC = 2
NS = 16
HALF = H // 2
CHUNK = 128
TILE_CHUNKS = 400
GROUP_PAIRS = TILE_CHUNKS // 16
TILE_ROWS = TILE_CHUNKS
EDGES_PAD = NS * TILE_CHUNKS * CHUNK
NODES_PAD = 50048
STRIPE = NODES_PAD // NS
BK = 1000


def _gelu(x):
    return 0.5 * x * (1.0 + lax.erf(x * (2.0 ** -0.5)))


def _pack_bf16(m):
    m16 = m.astype(jnp.bfloat16)
    lo = lax.bitcast_convert_type(m16[:, :16], jnp.uint16).astype(jnp.uint32)
    hi = lax.bitcast_convert_type(m16[:, 16:], jnp.uint16).astype(jnp.uint32)
    return lax.bitcast_convert_type(lo | (hi << 16), jnp.int32)


def _lstm_prep_body(nf_ref, wk_ref, wr_ref, b_ref, wp_ref, bp_ref, ew_ref,
                    x1_ref, ma_ref, mb_ref, s_ref):
    @pl.when(pl.program_id(0) == 0)
    def _():
        s_ref[...] = jnp.sum(ew_ref[...]).reshape(1, 1)

    wk = wk_ref[...]
    wr = wr_ref[...]
    b = b_ref[...]
    h = jnp.zeros((BK, U), jnp.float32)
    c = jnp.zeros((BK, U), jnp.float32)
    for t in range(T):
        xt = nf_ref[:, t * F:(t + 1) * F]
        z = (jnp.dot(xt, wk, preferred_element_type=jnp.float32)
             + jnp.dot(h, wr, preferred_element_type=jnp.float32) + b)
        i = jax.nn.sigmoid(z[:, 0:U])
        f = jax.nn.sigmoid(z[:, U:2 * U])
        g = jnp.tanh(z[:, 2 * U:3 * U])
        o = jax.nn.sigmoid(z[:, 3 * U:4 * U])
        c = f * c + i * g
        h = o * jnp.tanh(c)
    x1_ref[...] = h
    m = _gelu(jnp.dot(h * BN_SCALE, wp_ref[...],
                      preferred_element_type=jnp.float32) + bp_ref[...])
    ma_ref[...] = _pack_bf16(m[:, :HALF])
    mb_ref[...] = _pack_bf16(m[:, HALF:])


def _lstm_prep(nf, wk, wr, b, wp, bp, ew2d):
    grid = (N_NODES // BK,)
    return pl.pallas_call(
        _lstm_prep_body,
        grid=grid,
        in_specs=[
            pl.BlockSpec((BK, T * F), lambda i: (i, 0)),
            pl.BlockSpec((F, 4 * U), lambda i: (0, 0)),
            pl.BlockSpec((U, 4 * U), lambda i: (0, 0)),
            pl.BlockSpec((1, 4 * U), lambda i: (0, 0)),
            pl.BlockSpec((U, H), lambda i: (0, 0)),
            pl.BlockSpec((1, H), lambda i: (0, 0)),
            pl.BlockSpec((3125, 256), lambda i: (0, 0)),
        ],
        out_specs=[
            pl.BlockSpec((BK, U), lambda i: (i, 0)),
            pl.BlockSpec((BK, HALF // 2), lambda i: (i, 0)),
            pl.BlockSpec((BK, HALF // 2), lambda i: (i, 0)),
            pl.BlockSpec((1, 1), lambda i: (0, 0)),
        ],
        out_shape=[
            jax.ShapeDtypeStruct((N_NODES, U), jnp.float32),
            jax.ShapeDtypeStruct((N_NODES, HALF // 2), jnp.int32),
            jax.ShapeDtypeStruct((N_NODES, HALF // 2), jnp.int32),
            jax.ShapeDtypeStruct((1, 1), jnp.float32),
        ],
    )(nf, wk, wr, b, wp, bp, ew2d)


def _sc_conv_body(ma, mb, src2d, dst2d, w1d, outa, outb,
                  rbh0, rbh1, rbh2, rbh3, acc0, acc1,
                  gsrcA, gdstA, gwA, gsrcB, gdstB, gwB,
                  aggsm,
                  semG0, semG1, semG2, semG3,
                  semS0, semS1, semLA, semLB):
    c = lax.axis_index("c")
    s = lax.axis_index("s")
    base = s * STRIPE
    row0 = s * TILE_ROWS
    rbh = [rbh0, rbh1, rbh2, rbh3]
    acc = [acc0, acc1]
    semG = [semG0, semG1, semG2, semG3]
    semS = [semS0, semS1]

    def conv(m_hbm, out_hbm):
        @pl.loop(0, 128)
        def _(i):
            z16 = jnp.zeros((16,), jnp.float32)
            acc0[i, pl.ds(0, 16)] = z16
            acc0[i, pl.ds(16, 16)] = z16

        for q in range(24):
            pltpu.async_copy(acc0, aggsm.at[pl.ds(base + q * 128, 128), :],
                             semG0)
        pltpu.async_copy(acc0.at[pl.ds(0, 56), :],
                         aggsm.at[pl.ds(base + 3072, 56), :], semG0)
        for q in range(24):
            pltpu.make_async_copy(
                acc0, aggsm.at[pl.ds(base + q * 128, 128), :], semG0).wait()
        pltpu.make_async_copy(
            acc0.at[pl.ds(0, 56), :],
            aggsm.at[pl.ds(base + 3072, 56), :], semG0).wait()
        plsc.subcore_barrier()

        def fire_lin(g, gsrc, gdst, gw, semL):
            r = row0 + g * 8
            pltpu.async_copy(src2d.at[pl.ds(r, 8), :], gsrc, semL)
            pltpu.async_copy(dst2d.at[pl.ds(r, 8), :], gdst, semL)
            pltpu.async_copy(w1d.at[pl.ds(r * 128, 1024)], gw, semL)

        def wait_lin(g, gsrc, gdst, gw, semL):
            r = row0 + g * 8
            pltpu.make_async_copy(src2d.at[pl.ds(r, 8), :], gsrc, semL).wait()
            pltpu.make_async_copy(dst2d.at[pl.ds(r, 8), :], gdst, semL).wait()
            pltpu.make_async_copy(w1d.at[pl.ds(r * 128, 1024)], gw, semL).wait()

        def fire_gather(gsrc, ir, r):
            pltpu.async_copy(m_hbm.at[gsrc.at[ir]], rbh[r], semG[r])

        def wait_gather(gsrc, ir, r):
            pltpu.make_async_copy(m_hbm.at[gsrc.at[ir]], rbh[r],
                                  semG[r]).wait()

        def fire_scatter(gdst, ir, p):
            pltpu.async_copy(acc[p], aggsm.at[gdst.at[ir]], semS[p],
                             add=True)

        def wait_scatter(gdst, ir, p):
            pltpu.make_async_copy(acc[p], aggsm.at[gdst.at[ir]],
                                  semS[p]).wait()

        def scale(gw, off, r, p):
            rbuf = rbh[r]
            abuf = acc[p]

            @pl.loop(0, 16)
            def _(blk):
                b8 = blk * 8
                for t in range(8):
                    e = b8 + t
                    wv = plsc.load_gather(
                        gw, [jnp.full((16,), off + e, jnp.int32)])
                    row = plsc.bitcast(rbuf[e, pl.ds(0, 16)], jnp.bfloat16)
                    lo, hi = plsc.unpack(row, format=plsc.PackFormat.INTERLEAVED,
                                         preferred_element_type=jnp.float32)
                    abuf[e, pl.ds(0, 16)] = lo * wv
                    abuf[e, pl.ds(16, 16)] = hi * wv

        fire_lin(0, gsrcA, gdstA, gwA, semLA)
        wait_lin(0, gsrcA, gdstA, gwA, semLA)
        for i in range(3):
            fire_gather(gsrcA, i, i)

        @pl.loop(0, GROUP_PAIRS)
        def _(k):
            for i in range(16):
                if i < 8:
                    gsrc_i, gdst_i, gw_i = gsrcA, gdstA, gwA
                else:
                    gsrc_i, gdst_i, gw_i = gsrcB, gdstB, gwB
                r = i & 3
                p = i & 1
                ir = i & 7
                wait_gather(gsrc_i, ir, r)
                i2 = (i - 2) & 15
                pdst = gdstA if i2 < 8 else gdstB
                if i < 2:
                    @pl.when(k > 0)
                    def _():
                        wait_scatter(pdst, i2 & 7, p)
                else:
                    wait_scatter(pdst, i2 & 7, p)
                if i == 1:
                    fire_lin(2 * k + 1, gsrcB, gdstB, gwB, semLB)
                scale(gw_i, ir * 128, r, p)
                fire_scatter(gdst_i, ir, p)
                i3 = i + 3
                r3 = i3 & 3
                ir3 = i3 & 7
                if i3 < 8:
                    g3src = gsrcA
                elif i3 < 16:
                    g3src = gsrcB
                else:
                    g3src = gsrcA
                if i < 13:
                    if i == 5:
                        wait_lin(2 * k + 1, gsrcB, gdstB, gwB, semLB)
                    fire_gather(g3src, ir3, r3)
                    if i == 9:
                        @pl.when(k < GROUP_PAIRS - 1)
                        def _():
                            fire_lin(2 * k + 2, gsrcA, gdstA, gwA, semLA)
                else:
                    @pl.when(k < GROUP_PAIRS - 1)
                    def _():
                        if i == 13:
                            wait_lin(2 * k + 2, gsrcA, gdstA, gwA, semLA)
                        fire_gather(g3src, ir3, r3)

        wait_scatter(gdstB, 6, 0)
        wait_scatter(gdstB, 7, 1)

        plsc.subcore_barrier()

        for q in range(24):
            p = q & 1
            if q >= 2:
                pltpu.make_async_copy(
                    acc[p], out_hbm.at[pl.ds(base + (q - 2) * 128, 128), :],
                    semS[p]).wait()
            pltpu.sync_copy(aggsm.at[pl.ds(base + q * 128, 128), :], acc[p])
            pltpu.async_copy(acc[p],
                             out_hbm.at[pl.ds(base + q * 128, 128), :],
                             semS[p])
        pltpu.make_async_copy(
            acc[0], out_hbm.at[pl.ds(base + 22 * 128, 128), :], semS[0]).wait()
        pltpu.sync_copy(aggsm.at[pl.ds(base + 3072, 56), :],
                        acc0.at[pl.ds(0, 56), :])
        pltpu.async_copy(acc0.at[pl.ds(0, 56), :],
                         out_hbm.at[pl.ds(base + 3072, 56), :], semS[0])
        pltpu.make_async_copy(
            acc[1], out_hbm.at[pl.ds(base + 23 * 128, 128), :], semS[1]).wait()
        pltpu.make_async_copy(
            acc0.at[pl.ds(0, 56), :],
            out_hbm.at[pl.ds(base + 3072, 56), :], semS[0]).wait()

    @pl.when(c == 0)
    def _():
        conv(ma, outa)

    @pl.when(c == 1)
    def _():
        conv(mb, outb)


def _sc_conv(ma, mb, src2d, dst2d, w1d):
    mesh = plsc.VectorSubcoreMesh(core_axis_name="c", subcore_axis_name="s")
    f = pl.kernel(
        _sc_conv_body,
        out_type=(jax.ShapeDtypeStruct((NODES_PAD, HALF), jnp.float32),
                  jax.ShapeDtypeStruct((NODES_PAD, HALF), jnp.float32)),
        mesh=mesh,
        scratch_types=(
            [pltpu.VMEM((128, HALF // 2), jnp.int32) for _ in range(4)]
            + [pltpu.VMEM((128, HALF), jnp.float32) for _ in range(2)]
            + [pltpu.VMEM((8, 128), jnp.int32), pltpu.VMEM((8, 128), jnp.int32),
               pltpu.VMEM((1024,), jnp.float32)]
            + [pltpu.VMEM((8, 128), jnp.int32), pltpu.VMEM((8, 128), jnp.int32),
               pltpu.VMEM((1024,), jnp.float32)]
            + [pltpu.VMEM_SHARED((NODES_PAD, HALF), jnp.float32)]
            + [pltpu.SemaphoreType.DMA] * 8
        ),
        compiler_params=pltpu.CompilerParams(needs_layout_passes=False, use_tc_tiling_on_sc=False),
    )
    return f(ma, mb, src2d, dst2d, w1d)


def _update_body(x_ref, aa_ref, ab_ref, s_ref, wu_ref, bu_ref, wp_ref, bp_ref,
                 x2_ref, ma_ref, mb_ref):
    inv = BN_SCALE / s_ref[...]
    x = x_ref[...]
    wu = wu_ref[...]
    z = (BN_SCALE * jnp.dot(x, wu[:U, :], preferred_element_type=jnp.float32)
         + inv * jnp.dot(aa_ref[...], wu[U:U + HALF, :],
                         preferred_element_type=jnp.float32)
         + inv * jnp.dot(ab_ref[...], wu[U + HALF:, :],
                         preferred_element_type=jnp.float32)
         + bu_ref[...])
    out = _gelu(z)
    sq = jnp.maximum(jnp.sum(out * out, axis=-1, keepdims=True), 1e-12)
    x2 = out * lax.rsqrt(sq)
    x2_ref[...] = x2
    m = _gelu(jnp.dot(x2 * BN_SCALE, wp_ref[...],
                      preferred_element_type=jnp.float32) + bp_ref[...])
    ma_ref[...] = _pack_bf16(m[:, :HALF])
    mb_ref[...] = _pack_bf16(m[:, HALF:])


def _update(x1, aa, ab, s, wu, bu, wp, bp):
    grid = (N_NODES // BK,)
    return pl.pallas_call(
        _update_body,
        grid=grid,
        in_specs=[
            pl.BlockSpec((BK, U), lambda i: (i, 0)),
            pl.BlockSpec((BK, HALF), lambda i: (i, 0)),
            pl.BlockSpec((BK, HALF), lambda i: (i, 0)),
            pl.BlockSpec((1, 1), lambda i: (0, 0)),
            pl.BlockSpec((U + H, H), lambda i: (0, 0)),
            pl.BlockSpec((1, H), lambda i: (0, 0)),
            pl.BlockSpec((H, H), lambda i: (0, 0)),
            pl.BlockSpec((1, H), lambda i: (0, 0)),
        ],
        out_specs=[
            pl.BlockSpec((BK, H), lambda i: (i, 0)),
            pl.BlockSpec((BK, HALF // 2), lambda i: (i, 0)),
            pl.BlockSpec((BK, HALF // 2), lambda i: (i, 0)),
        ],
        out_shape=[
            jax.ShapeDtypeStruct((N_NODES, H), jnp.float32),
            jax.ShapeDtypeStruct((N_NODES, HALF // 2), jnp.int32),
            jax.ShapeDtypeStruct((N_NODES, HALF // 2), jnp.int32),
        ],
    )(x1, aa, ab, s, wu, bu, wp, bp)


def _sc_gather_body(x2, a2, b2, idx, ox, oa, ob, idxb, rx, ra, rb, sem):
    c = lax.axis_index("c")
    s = lax.axis_index("s")
    wid = s * NC + c
    per = BATCH // (NC * NS)
    base = wid * per
    pltpu.sync_copy(idx.at[pl.ds(base, per)], idxb)
    d1 = pltpu.async_copy(x2.at[idxb], rx, sem)
    d2 = pltpu.async_copy(a2.at[idxb], ra, sem)
    d3 = pltpu.async_copy(b2.at[idxb], rb, sem)
    d1.wait()
    d2.wait()
    d3.wait()
    pltpu.sync_copy(rx, ox.at[pl.ds(base, per), :])
    pltpu.sync_copy(ra, oa.at[pl.ds(base, per), :])
    pltpu.sync_copy(rb, ob.at[pl.ds(base, per), :])


def _sc_gather(x2, a2, b2, idx):
    per = BATCH // (NC * NS)
    mesh = plsc.VectorSubcoreMesh(core_axis_name="c", subcore_axis_name="s")
    f = pl.kernel(
        _sc_gather_body,
        out_type=(jax.ShapeDtypeStruct((BATCH, H), jnp.float32),
                  jax.ShapeDtypeStruct((BATCH, HALF), jnp.float32),
                  jax.ShapeDtypeStruct((BATCH, HALF), jnp.float32)),
        mesh=mesh,
        scratch_types=[
            pltpu.VMEM((per,), jnp.int32),
            pltpu.VMEM((per, H), jnp.float32),
            pltpu.VMEM((per, HALF), jnp.float32),
            pltpu.VMEM((per, HALF), jnp.float32),
            pltpu.SemaphoreType.DMA,
        ],
        compiler_params=pltpu.CompilerParams(needs_layout_passes=False, use_tc_tiling_on_sc=False),
    )
    return f(x2, a2, b2, idx)


def _final_body(x_ref, aa_ref, ab_ref, s_ref, wu_ref, bu_ref,
                wpo_ref, bpo_ref, wl_ref, bl_ref, out_ref):
    inv = BN_SCALE / s_ref[...]
    x = x_ref[...]
    wu = wu_ref[...]
    z = (BN_SCALE * jnp.dot(x, wu[:H, :], preferred_element_type=jnp.float32)
         + inv * jnp.dot(aa_ref[...], wu[H:H + HALF, :],
                         preferred_element_type=jnp.float32)
         + inv * jnp.dot(ab_ref[...], wu[H + HALF:, :],
                         preferred_element_type=jnp.float32)
         + bu_ref[...])
    out = _gelu(z)
    sq = jnp.maximum(jnp.sum(out * out, axis=-1, keepdims=True), 1e-12)
    x3 = out * lax.rsqrt(sq)
    x4 = _gelu(jnp.dot(x3 * BN_SCALE, wpo_ref[...],
                       preferred_element_type=jnp.float32) + bpo_ref[...])
    out_ref[...] = (jnp.dot(x4, wl_ref[...], preferred_element_type=jnp.float32)
                    + bl_ref[...])


def _final(xg, ag, bg, s, wu, bu, wpo, bpo, wl, bl):
    return pl.pallas_call(
        _final_body,
        out_shape=jax.ShapeDtypeStruct((BATCH, NUM_CLASSES), jnp.float32),
    )(xg, ag, bg, s, wu, bu, wpo, bpo, wl, bl)


def kernel(node_features, edge_weights, lstm_kernel, lstm_recurrent, lstm_bias,
           c1_prep_W, c1_prep_b, c1_upd_W, c1_upd_b,
           c2_prep_W, c2_prep_b, c2_upd_W, c2_upd_b,
           post_W, post_b, logits_W, logits_b, edges, input_node_indices):
    nf = node_features.reshape(N_NODES, T * F)
    dst = edges[0]
    src = edges[1]
    pad = EDGES_PAD - N_EDGES
    src2d = jnp.concatenate([src, jnp.zeros((pad,), jnp.int32)]).reshape(-1, 128)
    dst2d = jnp.concatenate([dst, jnp.zeros((pad,), jnp.int32)]).reshape(-1, 128)
    w1d = jnp.concatenate([edge_weights, jnp.zeros((pad,), jnp.float32)])

    wu1 = c1_upd_W
    wu2 = c2_upd_W

    x1, m1a, m1b, s = _lstm_prep(nf, lstm_kernel, lstm_recurrent,
                                 lstm_bias.reshape(1, -1),
                                 c1_prep_W, c1_prep_b.reshape(1, -1),
                                 edge_weights.reshape(3125, 256))
    a1a, a1b = _sc_conv(m1a, m1b, src2d, dst2d, w1d)
    x2, m2a, m2b = _update(x1, a1a, a1b, s, wu1,
                           c1_upd_b.reshape(1, -1),
                           c2_prep_W, c2_prep_b.reshape(1, -1))
    a2a, a2b = _sc_conv(m2a, m2b, src2d, dst2d, w1d)
    xg, ag, bg = _sc_gather(x2, a2a, a2b, input_node_indices)
    return _final(xg, ag, bg, s, wu2, c2_upd_b.reshape(1, -1),
                  post_W, post_b.reshape(1, -1),
                  logits_W, logits_b.reshape(1, -1))
